# Initial kernel scaffold; baseline (speedup 1.0000x reference)
import jax
import jax.numpy as jnp
from jax.experimental import pallas as pl


def kernel(h, edge_index, edge_type, edge_time, norm, emb_rel, emb_time, weight_neighbor, loop_weight, evolve_loop_weight):
    raise NotImplementedError("write your pallas kernel here")


# speedup vs baseline: 5.8340x; 5.8340x over previous
import functools

import jax
import jax.numpy as jnp
from jax import lax
from jax.experimental import pallas as pl
from jax.experimental.pallas import tpu as pltpu
from jax.experimental.pallas import tpu_sc as plsc

N = 10000
E = 320000
D = 128
NR = 200
NT = 366

NC = 2
NS = 16
NW = NC * NS
EPW = E // NW
C = 80
NCHUNK = EPW // C
NPAD = 10240
RPT = NPAD // NS


def _sc_body(h_hbm, src_hbm, dst_hbm, et_hbm, tt_hbm, rel_hbm, time_hbm,
             pa_out, deg_out,
             pa_s, rel_s, time_s,
             src_v, dst_v, et_v, tt_v,
             hbuf, relbuf, timebuf, hist_v,
             sem_h, sem_r, sem_t):
    c = lax.axis_index("c")
    s = lax.axis_index("s")

    z16 = jnp.zeros((16,), jnp.float32)
    o16 = jnp.ones((16,), jnp.float32)

    def zhist(i, carry):
        hist_v[pl.ds(i * 16, 16)] = z16
        return carry
    lax.fori_loop(0, NPAD // 16, zhist, 0)

    def zrow(i, carry):
        for d8 in range(8):
            hbuf[i, pl.ds(d8 * 16, 16)] = z16
        return carry
    lax.fori_loop(0, C, zrow, 0)

    base_row = s * RPT
    for j in range(RPT // C):
        pltpu.sync_copy(hbuf, pa_s.at[pl.ds(base_row + j * C, C), :])

    @pl.when(s == 0)
    def _():
        pltpu.sync_copy(rel_hbm, rel_s)
        pltpu.sync_copy(time_hbm, time_s)

    plsc.subcore_barrier()

    ebase = (c * NS + s) * EPW

    def chunk(i, carry):
        off = ebase + i * C
        pltpu.sync_copy(src_hbm.at[pl.ds(off, C)], src_v)
        pltpu.sync_copy(et_hbm.at[pl.ds(off, C)], et_v)
        pltpu.sync_copy(tt_hbm.at[pl.ds(off, C)], tt_v)
        pltpu.sync_copy(dst_hbm.at[pl.ds(off, C)], dst_v)
        cp_h = pltpu.async_copy(h_hbm.at[src_v], hbuf, sem_h)
        cp_r = pltpu.async_copy(rel_s.at[et_v], relbuf, sem_r)
        cp_t = pltpu.async_copy(time_s.at[tt_v], timebuf, sem_t)
        cp_h.wait()
        cp_r.wait()
        cp_t.wait()

        def frow(r, inner):
            for d8 in range(8):
                sl = pl.ds(d8 * 16, 16)
                hbuf[r, sl] = hbuf[r, sl] + relbuf[r, sl] * timebuf[r, sl]
            return inner
        lax.fori_loop(0, C, frow, 0)

        pltpu.sync_copy(hbuf, pa_s.at[dst_v], add=True)

        for k in range(C // 16):
            idx16 = dst_v[pl.ds(k * 16, 16)]
            plsc.store_scatter(hist_v, [idx16], o16)
        return carry

    lax.fori_loop(0, NCHUNK, chunk, 0)

    plsc.subcore_barrier()

    pltpu.sync_copy(pa_s.at[pl.ds(base_row, RPT), :],
                    pa_out.at[c, pl.ds(base_row, RPT), :])
    pltpu.sync_copy(hist_v, deg_out.at[c * NS + s, :])


_sc_call = pl.kernel(
    _sc_body,
    out_type=[
        jax.ShapeDtypeStruct((NC, NPAD, D), jnp.float32),
        jax.ShapeDtypeStruct((NW, NPAD), jnp.float32),
    ],
    mesh=plsc.VectorSubcoreMesh(core_axis_name="c", subcore_axis_name="s"),
    compiler_params=pltpu.CompilerParams(needs_layout_passes=False),
    scratch_types=[
        pltpu.VMEM_SHARED((NPAD, D), jnp.float32),
        pltpu.VMEM_SHARED((NR, D), jnp.float32),
        pltpu.VMEM_SHARED((NT, D), jnp.float32),
        pltpu.VMEM((C,), jnp.int32),
        pltpu.VMEM((C,), jnp.int32),
        pltpu.VMEM((C,), jnp.int32),
        pltpu.VMEM((C,), jnp.int32),
        pltpu.VMEM((C, D), jnp.float32),
        pltpu.VMEM((C, D), jnp.float32),
        pltpu.VMEM((C, D), jnp.float32),
        pltpu.VMEM((NPAD,), jnp.float32),
        pltpu.SemaphoreType.DMA,
        pltpu.SemaphoreType.DMA,
        pltpu.SemaphoreType.DMA,
    ],
)


BLK = 1000


def _tc_body(pa_ref, deg_ref, h_ref, norm_ref, wn_ref, wl_ref, we_ref, o_ref):
    pa = pa_ref[0] + pa_ref[1]
    deg = jnp.sum(deg_ref[...], axis=1)[:, None]
    hb = h_ref[...]
    agg = jnp.dot(pa, wn_ref[...], preferred_element_type=jnp.float32)
    lm = jnp.dot(hb, wl_ref[...], preferred_element_type=jnp.float32)
    le = jnp.dot(hb, we_ref[...], preferred_element_type=jnp.float32)
    o_ref[...] = agg * norm_ref[...] + jnp.where(deg > 0.0, lm, le)


def _tc_call(pa, deg, h, norm, wn, wl, we):
    return pl.pallas_call(
        _tc_body,
        grid=(N // BLK,),
        in_specs=[
            pl.BlockSpec((NC, BLK, D), lambda i: (0, i, 0)),
            pl.BlockSpec((BLK, NW), lambda i: (i, 0)),
            pl.BlockSpec((BLK, D), lambda i: (i, 0)),
            pl.BlockSpec((BLK, 1), lambda i: (i, 0)),
            pl.BlockSpec((D, D), lambda i: (0, 0)),
            pl.BlockSpec((D, D), lambda i: (0, 0)),
            pl.BlockSpec((D, D), lambda i: (0, 0)),
        ],
        out_specs=pl.BlockSpec((BLK, D), lambda i: (i, 0)),
        out_shape=jax.ShapeDtypeStruct((N, D), jnp.float32),
    )(pa, deg, h, norm, wn, wl, we)


def kernel(h, edge_index, edge_type, edge_time, norm, emb_rel, emb_time,
           weight_neighbor, loop_weight, evolve_loop_weight):
    src = edge_index[0].astype(jnp.int32)
    dst = edge_index[1].astype(jnp.int32)
    et = edge_type.astype(jnp.int32)
    tt = edge_time.astype(jnp.int32)
    pa, deg = _sc_call(h, src, dst, et, tt, emb_rel, emb_time)
    return _tc_call(pa, deg.T, h, norm, weight_neighbor, loop_weight,
                    evolve_loop_weight)
